# baseline (speedup 1.0000x reference)
import functools
import numpy as np
import jax
import jax.numpy as jnp
from jax import lax
from jax.experimental import pallas as pl
from jax.experimental.pallas import tpu as pltpu, tpu_sc as plsc

_SZ = 4096
_RT = 3264
_NSC = _SZ - _RT
_NW = 32
_RPW = _NSC // _NW

_K0 = np.uint32(0x375F238F)
_K1 = np.uint32(0xCDDB151D)
_K2 = np.uint32(int(_K0) ^ int(_K1) ^ 0x1BD11BDA)
_ROT = ((13, 15, 26, 6), (17, 29, 16, 24))
_KEYS = tuple(
    (ka, np.uint32((int(kb) + i + 1) & 0xFFFFFFFF))
    for i, (ka, kb) in enumerate(
        ((_K1, _K2), (_K2, _K0), (_K0, _K1), (_K1, _K2), (_K2, _K0))
    )
)


def _bits(x1):
    x0 = jnp.full_like(x1, _K0)
    x1 = x1 + _K1
    for i, (ka, kb_inc) in enumerate(_KEYS):
        for r in _ROT[i % 2]:
            x0 = x0 + x1
            x1 = (x1 << r) | (x1 >> (32 - r))
            x1 = x0 ^ x1
        x0 = x0 + ka
        x1 = x1 + kb_inc
    return x0 ^ x1


def _w(bits):
    f = jax.lax.bitcast_convert_type(
        (bits >> 9) | np.uint32(0x3F800000), jnp.float32) - 1.0
    return -jnp.log(f)


def _gumbel_kernel(x_ref, o_ref):
    i = pl.program_id(0)
    br = x_ref.shape[0]
    c = x_ref.shape[2]
    l0 = x_ref[:, 0, :]
    l1 = x_ref[:, 1, :]
    row = jax.lax.broadcasted_iota(jnp.uint32, (br, c), 0)
    col = jax.lax.broadcasted_iota(jnp.uint32, (br, c), 1)
    base = (row + (i * br).astype(jnp.uint32)) * np.uint32(2 * c) + col * np.uint32(2)
    w0 = _w(_bits(base))
    w1 = _w(_bits(base + np.uint32(1)))
    t = jnp.exp(l0 - l1)
    o_ref[...] = jnp.where(w0 <= w1 * t, jnp.float32(1.0), jnp.float32(0.0))


_LN2 = np.float32(0.6931471805599453)
_SQRT2 = np.float32(1.4142135623730951)


def _w_sc(bits):
    f = lax.bitcast_convert_type(
        (bits >> 9) | np.uint32(0x3F800000), jnp.float32) - np.float32(1.0)
    f = jnp.maximum(f, np.float32(1e-20))
    bf = lax.bitcast_convert_type(f, jnp.uint32)
    e = ((bf >> 23) & np.uint32(0xFF)).astype(jnp.int32) - 127
    m = lax.bitcast_convert_type(
        (bf & np.uint32(0x007FFFFF)) | np.uint32(0x3F800000), jnp.float32)
    big = m > _SQRT2
    m = jnp.where(big, m * np.float32(0.5), m)
    e = (e + jnp.where(big, 1, 0)).astype(jnp.float32)
    z = (m - np.float32(1.0)) / (m + np.float32(1.0))
    s = z * z
    ln_m = z * (np.float32(2.0) + s * (np.float32(2.0 / 3.0) + s * (
        np.float32(2.0 / 5.0) + s * np.float32(2.0 / 7.0))))
    return -(e * _LN2 + ln_m)


def _sc_body(x_hbm, o_hbm, xin0, xin1, xout):
    wid = lax.axis_index("s") * 2 + lax.axis_index("c")
    iota = lax.broadcasted_iota(jnp.uint32, (16,), 0)

    def row_loop(k, carry):
        rg = _RT + wid * _RPW + k
        pltpu.sync_copy(x_hbm.at[rg, 0], xin0)
        pltpu.sync_copy(x_hbm.at[rg, 1], xin1)

        def chunk(q, carry2):
            cb = q * 64
            row_base = jnp.uint32(rg) * np.uint32(8192)
            for j in range(4):
                off = cb + j * 16
                l0j = xin0[pl.ds(off, 16)]
                l1j = xin1[pl.ds(off, 16)]
                cnt = np.uint32(2) * (jnp.uint32(off) + iota) + row_base
                w0 = _w_sc(_bits(cnt))
                w1 = _w_sc(_bits(cnt + np.uint32(1)))
                res = jnp.where(w0 <= w1 * jnp.exp(l0j - l1j),
                                np.float32(1.0), np.float32(0.0))
                xout[pl.ds(off, 16)] = res
            return carry2

        lax.fori_loop(0, 64, chunk, 0)
        orow = wid * _RPW + k
        pltpu.sync_copy(xout, o_hbm.at[pl.ds(orow * 4096, 4096)])
        return carry

    lax.fori_loop(0, _RPW, row_loop, 0)


def _sc_call(t):
    mesh = plsc.VectorSubcoreMesh(core_axis_name="c", subcore_axis_name="s")
    k = functools.partial(
        pl.kernel, mesh=mesh,
        out_type=jax.ShapeDtypeStruct((_NSC * _SZ,), jnp.float32),
        scratch_types=[
            pltpu.VMEM((_SZ,), jnp.float32),
            pltpu.VMEM((_SZ,), jnp.float32),
            pltpu.VMEM((_SZ,), jnp.float32),
        ],
    )(_sc_body)
    return k(t)


def kernel(gen_matrix):
    t = jnp.transpose(gen_matrix, (0, 2, 1))
    sc_flat = _sc_call(t)
    br = 64
    tc = pl.pallas_call(
        _gumbel_kernel,
        grid=(_RT // br,),
        in_specs=[
            pl.BlockSpec((br, 2, _SZ), lambda i: (i, 0, 0)),
        ],
        out_specs=pl.BlockSpec((br, _SZ), lambda i: (i, 0)),
        out_shape=jax.ShapeDtypeStruct((_SZ, _SZ), jnp.float32),
    )(t)
    return lax.dynamic_update_slice(tc, sc_flat.reshape(_NSC, _SZ), (_RT, 0))


# speedup vs baseline: 1.0291x; 1.0291x over previous
import functools
import numpy as np
import jax
import jax.numpy as jnp
from jax import lax
from jax.experimental import pallas as pl
from jax.experimental.pallas import tpu as pltpu, tpu_sc as plsc

_SZ = 4096
_RT = 3328
_NSC = _SZ - _RT
_NW = 32
_RPW = _NSC // _NW

_K0 = np.uint32(0x375F238F)
_K1 = np.uint32(0xCDDB151D)
_K2 = np.uint32(int(_K0) ^ int(_K1) ^ 0x1BD11BDA)
_ROT = ((13, 15, 26, 6), (17, 29, 16, 24))
_KEYS = tuple(
    (ka, np.uint32((int(kb) + i + 1) & 0xFFFFFFFF))
    for i, (ka, kb) in enumerate(
        ((_K1, _K2), (_K2, _K0), (_K0, _K1), (_K1, _K2), (_K2, _K0))
    )
)


def _bits(x1):
    x0 = jnp.full_like(x1, _K0)
    x1 = x1 + _K1
    for i, (ka, kb_inc) in enumerate(_KEYS):
        for r in _ROT[i % 2]:
            x0 = x0 + x1
            x1 = (x1 << r) | (x1 >> (32 - r))
            x1 = x0 ^ x1
        x0 = x0 + ka
        x1 = x1 + kb_inc
    return x0 ^ x1


def _w(bits):
    f = jax.lax.bitcast_convert_type(
        (bits >> 9) | np.uint32(0x3F800000), jnp.float32) - 1.0
    return -jnp.log(f)


def _gumbel_kernel(x_ref, o_ref):
    i = pl.program_id(0)
    br = x_ref.shape[0]
    c = x_ref.shape[2]
    l0 = x_ref[:, 0, :]
    l1 = x_ref[:, 1, :]
    row = jax.lax.broadcasted_iota(jnp.uint32, (br, c), 0)
    col = jax.lax.broadcasted_iota(jnp.uint32, (br, c), 1)
    base = (row + (i * br).astype(jnp.uint32)) * np.uint32(2 * c) + col * np.uint32(2)
    w0 = _w(_bits(base))
    w1 = _w(_bits(base + np.uint32(1)))
    t = jnp.exp(l0 - l1)
    o_ref[...] = jnp.where(w0 <= w1 * t, jnp.float32(1.0), jnp.float32(0.0))


_LN2 = np.float32(0.6931471805599453)
_SQRT2 = np.float32(1.4142135623730951)


def _w_sc(bits):
    f = lax.bitcast_convert_type(
        (bits >> 9) | np.uint32(0x3F800000), jnp.float32) - np.float32(1.0)
    f = jnp.maximum(f, np.float32(1e-20))
    bf = lax.bitcast_convert_type(f, jnp.uint32)
    e = ((bf >> 23) & np.uint32(0xFF)).astype(jnp.int32) - 127
    m = lax.bitcast_convert_type(
        (bf & np.uint32(0x007FFFFF)) | np.uint32(0x3F800000), jnp.float32)
    big = m > _SQRT2
    m = jnp.where(big, m * np.float32(0.5), m)
    e = (e + jnp.where(big, 1, 0)).astype(jnp.float32)
    z = (m - np.float32(1.0)) / (m + np.float32(1.0))
    s = z * z
    ln_m = z * (np.float32(2.0) + s * (np.float32(2.0 / 3.0) + s * (
        np.float32(2.0 / 5.0) + s * np.float32(2.0 / 7.0))))
    return -(e * _LN2 + ln_m)


def _sc_body(x_hbm, o_hbm, xin0, xin1, xout):
    wid = lax.axis_index("s") * 2 + lax.axis_index("c")
    iota = lax.broadcasted_iota(jnp.uint32, (16,), 0)

    def row_loop(k, carry):
        rg = _RT + wid * _RPW + k
        pltpu.sync_copy(x_hbm.at[rg, 0], xin0)
        pltpu.sync_copy(x_hbm.at[rg, 1], xin1)

        def chunk(q, carry2):
            cb = q * 64
            row_base = jnp.uint32(rg) * np.uint32(8192)
            for j in range(4):
                off = cb + j * 16
                l0j = xin0[pl.ds(off, 16)]
                l1j = xin1[pl.ds(off, 16)]
                cnt = np.uint32(2) * (jnp.uint32(off) + iota) + row_base
                w0 = _w_sc(_bits(cnt))
                w1 = _w_sc(_bits(cnt + np.uint32(1)))
                res = jnp.where(w0 <= w1 * jnp.exp(l0j - l1j),
                                np.float32(1.0), np.float32(0.0))
                xout[pl.ds(off, 16)] = res
            return carry2

        lax.fori_loop(0, 64, chunk, 0)
        orow = wid * _RPW + k
        pltpu.sync_copy(xout, o_hbm.at[pl.ds(orow * 4096, 4096)])
        return carry

    lax.fori_loop(0, _RPW, row_loop, 0)


def _sc_call(t):
    mesh = plsc.VectorSubcoreMesh(core_axis_name="c", subcore_axis_name="s")
    k = functools.partial(
        pl.kernel, mesh=mesh,
        out_type=jax.ShapeDtypeStruct((_NSC * _SZ,), jnp.float32),
        scratch_types=[
            pltpu.VMEM((_SZ,), jnp.float32),
            pltpu.VMEM((_SZ,), jnp.float32),
            pltpu.VMEM((_SZ,), jnp.float32),
        ],
    )(_sc_body)
    return k(t)


def kernel(gen_matrix):
    t = jnp.transpose(gen_matrix, (0, 2, 1))
    sc_flat = _sc_call(t)
    br = 128
    tc = pl.pallas_call(
        _gumbel_kernel,
        grid=(_RT // br,),
        in_specs=[
            pl.BlockSpec((br, 2, _SZ), lambda i: (i, 0, 0)),
        ],
        out_specs=pl.BlockSpec((br, _SZ), lambda i: (i, 0)),
        out_shape=jax.ShapeDtypeStruct((_SZ, _SZ), jnp.float32),
    )(t)
    return lax.dynamic_update_slice(tc, sc_flat.reshape(_NSC, _SZ), (_RT, 0))
